# baseline (device time: 30889 ns/iter reference)
import jax
import jax.numpy as jnp
from jax import lax
from jax.experimental import pallas as pl
from jax.experimental.pallas import tpu as pltpu

Z = 4
CHUNK_ROWS = (32, 48, 64, 48, 32, 16, 16)
C = len(CHUNK_ROWS)
CHUNK_OFF = tuple(sum(CHUNK_ROWS[:i]) for i in range(C))


def kernel(x, pi):
    _, m, n = x.shape
    quarter = m // 4
    assert sum(CHUNK_ROWS) == quarter

    def body(x_ref, pi_ref, out_ref, send_buf, zq_buf, xq_buf, yq_buf, dq_buf,
             zs, zr, xs, xr, ys, yr, ds, dr):
        my_x = lax.axis_index("x")
        my_y = lax.axis_index("y")
        my_z = lax.axis_index("z")

        dst_z = pi_ref[my_z]
        src_z = jnp.int32(0)
        for j in range(Z):
            src_z = jnp.where(pi_ref[j] == my_z, jnp.int32(j), src_z)

        xn = 1 - my_x
        yp = my_y + 1 - 2 * (my_y % 2)
        q_me = 2 * my_x + (my_y % 2)

        x_dev = (xn, my_y, my_z)
        y_dev = (my_x, yp, my_z)
        d_dev = (xn, yp, my_z)
        peers = ((my_x, my_y, src_z), (my_x, my_y, dst_z), x_dev, y_dev, d_dev)

        my_off = q_me * quarter

        sl0 = pl.ds(CHUNK_OFF[0], CHUNK_ROWS[0])
        send_buf[sl0, :] = x_ref[
            0, pl.ds(my_off + CHUNK_OFF[0], CHUNK_ROWS[0]), :
        ].astype(jnp.bfloat16)

        barrier = pltpu.get_barrier_semaphore()
        for dev in peers:
            pl.semaphore_signal(
                barrier, inc=1, device_id=dev,
                device_id_type=pl.DeviceIdType.MESH,
            )
        pl.semaphore_wait(barrier, 5)

        z_rdmas = []
        for c in range(C):
            sl = pl.ds(CHUNK_OFF[c], CHUNK_ROWS[c])
            if c > 0:
                send_buf[sl, :] = x_ref[
                    0, pl.ds(my_off + CHUNK_OFF[c], CHUNK_ROWS[c]), :
                ].astype(jnp.bfloat16)
            r = pltpu.make_async_remote_copy(
                src_ref=send_buf.at[sl],
                dst_ref=zq_buf.at[sl],
                send_sem=zs.at[c],
                recv_sem=zr.at[c],
                device_id=(my_x, my_y, dst_z),
                device_id_type=pl.DeviceIdType.MESH,
            )
            r.start()
            z_rdmas.append(r)

        swap_rdmas = []
        for c in range(C):
            sl = pl.ds(CHUNK_OFF[c], CHUNK_ROWS[c])
            z_rdmas[c].wait_recv()
            chunk_rdmas = []
            for dev, dst_buf, s_sem, r_sem in (
                (x_dev, xq_buf, xs, xr),
                (y_dev, yq_buf, ys, yr),
                (d_dev, dq_buf, ds, dr),
            ):
                r2 = pltpu.make_async_remote_copy(
                    src_ref=zq_buf.at[sl],
                    dst_ref=dst_buf.at[sl],
                    send_sem=s_sem.at[c],
                    recv_sem=r_sem.at[c],
                    device_id=dev,
                    device_id_type=pl.DeviceIdType.MESH,
                )
                r2.start()
                chunk_rdmas.append(r2)
            swap_rdmas.append(chunk_rdmas)
            out_ref[0, pl.ds(my_off + CHUNK_OFF[c], CHUNK_ROWS[c]), :] = (
                zq_buf[sl, :].astype(jnp.float32)
            )

        for c in range(C):
            sl = pl.ds(CHUNK_OFF[c], CHUNK_ROWS[c])
            for k, buf in ((2, xq_buf), (1, yq_buf), (3, dq_buf)):
                q_peer = jnp.bitwise_xor(q_me, k)
                swap_rdmas[c][{2: 0, 1: 1, 3: 2}[k]].wait_recv()
                out_ref[
                    0, pl.ds(q_peer * quarter + CHUNK_OFF[c], CHUNK_ROWS[c]), :
                ] = buf[sl, :].astype(jnp.float32)

        for c in range(C):
            z_rdmas[c].wait_send()
            for r2 in swap_rdmas[c]:
                r2.wait_send()


    return pl.pallas_call(
        body,
        out_shape=jax.ShapeDtypeStruct((1, m, n), jnp.float32),
        in_specs=[
            pl.BlockSpec(memory_space=pltpu.VMEM),
            pl.BlockSpec(memory_space=pltpu.SMEM),
        ],
        out_specs=pl.BlockSpec(memory_space=pltpu.VMEM),
        scratch_shapes=[
            pltpu.VMEM((quarter, n), jnp.bfloat16),
            pltpu.VMEM((quarter, n), jnp.bfloat16),
            pltpu.VMEM((quarter, n), jnp.bfloat16),
            pltpu.VMEM((quarter, n), jnp.bfloat16),
            pltpu.VMEM((quarter, n), jnp.bfloat16),
            pltpu.SemaphoreType.DMA((C,)),
            pltpu.SemaphoreType.DMA((C,)),
            pltpu.SemaphoreType.DMA((C,)),
            pltpu.SemaphoreType.DMA((C,)),
            pltpu.SemaphoreType.DMA((C,)),
            pltpu.SemaphoreType.DMA((C,)),
            pltpu.SemaphoreType.DMA((C,)),
            pltpu.SemaphoreType.DMA((C,)),
        ],
        compiler_params=pltpu.CompilerParams(collective_id=0),
    )(x, pi)
